# Initial kernel scaffold; baseline (speedup 1.0000x reference)
import jax
import jax.numpy as jnp
from jax.experimental import pallas as pl


def kernel(x, edge_index, W1, a_src1, a_dst1, b1, W2, a_src2, a_dst2, b2, gamma1, beta1, gamma2, beta2):
    raise NotImplementedError("write your pallas kernel here")


# speedup vs baseline: 76.4780x; 76.4780x over previous
import functools

import jax
import jax.numpy as jnp
from jax import lax
from jax.experimental import pallas as pl
from jax.experimental.pallas import tpu as pltpu
from jax.experimental.pallas import tpu_sc as plsc

N_NODES = 10000
DIM = 128
HEADS = 8
CHAN = 16
NP = 10240
BL = 128
NC, NS = 2, 16
NW = NC * NS
BN = 1024
ROWS_PER_TILE = NP // NS


def _hi_dot(a, b):
    return jnp.dot(a, b, preferred_element_type=jnp.float32,
                   precision=lax.Precision.HIGHEST)


def _tc1_body(x_ref, w_ref, am_ref, h_ref, asrc_ref, adst_ref):
    h = _hi_dot(x_ref[...], w_ref[...])
    h_ref[...] = h
    ab = _hi_dot(h, am_ref[...])
    asrc_ref[...] = ab[:, :CHAN]
    adst_ref[...] = ab[:, CHAN:]


def _tc1(xp, W, AM):
    return pl.pallas_call(
        _tc1_body,
        grid=(NP // BN,),
        in_specs=[
            pl.BlockSpec((BN, DIM), lambda i: (i, 0)),
            pl.BlockSpec((DIM, DIM), lambda i: (0, 0)),
            pl.BlockSpec((DIM, 2 * CHAN), lambda i: (0, 0)),
        ],
        out_specs=[
            pl.BlockSpec((BN, DIM), lambda i: (i, 0)),
            pl.BlockSpec((BN, CHAN), lambda i: (i, 0)),
            pl.BlockSpec((BN, CHAN), lambda i: (i, 0)),
        ],
        out_shape=[
            jax.ShapeDtypeStruct((NP, DIM), jnp.float32),
            jax.ShapeDtypeStruct((NP, CHAN), jnp.float32),
            jax.ShapeDtypeStruct((NP, CHAN), jnp.float32),
        ],
    )(xp, W, AM)


def _tc2_body(n0_ref, n1_ref, d0_ref, d1_ref, b_ref, w_ref, am_ref,
              h_ref, asrc_ref, adst_ref):
    num = n0_ref[...] + n1_ref[...]
    den = d0_ref[...] + d1_ref[...]
    cols = []
    for k in range(HEADS):
        cols.append(num[:, CHAN * k:CHAN * (k + 1)] /
                    (den[:, k:k + 1] + 1e-16))
    x1 = jnp.concatenate(cols, axis=1)
    x1 = jnp.maximum(x1 + b_ref[...], 0.0)
    h = _hi_dot(x1, w_ref[...])
    h_ref[...] = h
    ab = _hi_dot(h, am_ref[...])
    asrc_ref[...] = ab[:, :CHAN]
    adst_ref[...] = ab[:, CHAN:]


def _tc2(numer, denom, b1, W2, AM2):
    nblk = NP // BN
    return pl.pallas_call(
        _tc2_body,
        grid=(nblk,),
        in_specs=[
            pl.BlockSpec((BN, DIM), lambda i: (i, 0)),
            pl.BlockSpec((BN, DIM), lambda i, n=nblk: (n + i, 0)),
            pl.BlockSpec((BN, CHAN), lambda i: (i, 0)),
            pl.BlockSpec((BN, CHAN), lambda i, n=nblk: (n + i, 0)),
            pl.BlockSpec((1, DIM), lambda i: (0, 0)),
            pl.BlockSpec((DIM, DIM), lambda i: (0, 0)),
            pl.BlockSpec((DIM, 2 * CHAN), lambda i: (0, 0)),
        ],
        out_specs=[
            pl.BlockSpec((BN, DIM), lambda i: (i, 0)),
            pl.BlockSpec((BN, CHAN), lambda i: (i, 0)),
            pl.BlockSpec((BN, CHAN), lambda i: (i, 0)),
        ],
        out_shape=[
            jax.ShapeDtypeStruct((NP, DIM), jnp.float32),
            jax.ShapeDtypeStruct((NP, CHAN), jnp.float32),
            jax.ShapeDtypeStruct((NP, CHAN), jnp.float32),
        ],
    )(numer, numer, denom, denom, b1, W2, AM2)


def _tc3_body(n0_ref, n1_ref, d0_ref, d1_ref, b_ref, out_ref):
    num = n0_ref[...] + n1_ref[...]
    den = d0_ref[...] + d1_ref[...]
    acc = num[:, :CHAN] / (den[:, 0:1] + 1e-16)
    for k in range(1, HEADS):
        acc = acc + num[:, CHAN * k:CHAN * (k + 1)] / (den[:, k:k + 1] + 1e-16)
    out_ref[...] = acc * (1.0 / HEADS) + b_ref[...]


def _tc3(numer, denom, b2):
    nblk = NP // BN
    return pl.pallas_call(
        _tc3_body,
        grid=(nblk,),
        in_specs=[
            pl.BlockSpec((BN, DIM), lambda i: (i, 0)),
            pl.BlockSpec((BN, DIM), lambda i, n=nblk: (n + i, 0)),
            pl.BlockSpec((BN, CHAN), lambda i: (i, 0)),
            pl.BlockSpec((BN, CHAN), lambda i, n=nblk: (n + i, 0)),
            pl.BlockSpec((1, CHAN), lambda i: (0, 0)),
        ],
        out_specs=pl.BlockSpec((BN, CHAN), lambda i: (i, 0)),
        out_shape=jax.ShapeDtypeStruct((NP, CHAN), jnp.float32),
    )(numer, numer, denom, denom, b2)


def _make_sc_aggregate(chunks_per_tile):
    mesh = plsc.VectorSubcoreMesh(core_axis_name="c", subcore_axis_name="s",
                                  num_cores=NC, num_subcores=NS)

    @functools.partial(
        pl.kernel,
        out_type=(
            jax.ShapeDtypeStruct((NC * NP, DIM), jnp.float32),
            jax.ShapeDtypeStruct((NC * NP, CHAN), jnp.float32),
        ),
        mesh=mesh,
        compiler_params=pltpu.CompilerParams(use_tc_tiling_on_sc=False),
        scratch_types=(
            pltpu.VMEM_SHARED((NP, DIM), jnp.float32),
            pltpu.VMEM_SHARED((NP, CHAN), jnp.float32),
            pltpu.VMEM((BL,), jnp.int32),
            pltpu.VMEM((BL,), jnp.int32),
            pltpu.VMEM((BL, CHAN), jnp.float32),
            pltpu.VMEM((BL, CHAN), jnp.float32),
            pltpu.VMEM((BL, CHAN), jnp.float32),
            pltpu.VMEM((BL, DIM), jnp.float32),
            pltpu.SemaphoreType.DMA,
            pltpu.SemaphoreType.DMA,
            pltpu.SemaphoreType.DMA,
        ),
    )
    def sc_agg(h_hbm, asrc_hbm, adst_hbm, src_hbm, dst_hbm,
               numer_hbm, denom_hbm,
               out_acc, den_acc, src_buf, dst_buf, asr_buf, adr_buf,
               w_buf, h_buf, sem_h, sem_a, sem_b):
        cid = lax.axis_index("c")
        sid = lax.axis_index("s")
        wid = cid * NS + sid
        base = sid * ROWS_PER_TILE

        zero16 = jnp.zeros((16,), jnp.float32)

        def zrow(i, carry):
            for k in range(DIM // 16):
                h_buf[i, pl.ds(16 * k, 16)] = zero16
            w_buf[i, :] = zero16
            return carry

        lax.fori_loop(0, BL, zrow, 0)
        for j in range(ROWS_PER_TILE // BL):
            pltpu.sync_copy(h_buf, out_acc.at[pl.ds(base + j * BL, BL)])
            pltpu.sync_copy(w_buf, den_acc.at[pl.ds(base + j * BL, BL)])
        plsc.subcore_barrier()

        def chunk(j, carry):
            off = (wid * chunks_per_tile + j) * BL
            pltpu.sync_copy(src_hbm.at[pl.ds(off, BL)], src_buf)
            pltpu.sync_copy(dst_hbm.at[pl.ds(off, BL)], dst_buf)
            cp_h = pltpu.async_copy(h_hbm.at[src_buf], h_buf, sem_h)
            cp_a = pltpu.async_copy(asrc_hbm.at[src_buf], asr_buf, sem_a)
            cp_b = pltpu.async_copy(adst_hbm.at[dst_buf], adr_buf, sem_b)
            cp_a.wait()
            cp_b.wait()

            def wloop(e, c2):
                s = asr_buf[e, :] + adr_buf[e, :]
                w_buf[e, :] = jnp.exp(jnp.where(s > 0.0, s, 0.2 * s))
                return c2

            lax.fori_loop(0, BL, wloop, 0)
            pltpu.sync_copy(w_buf, den_acc.at[dst_buf], add=True)
            cp_h.wait()

            def mloop(e, c2):
                wv = w_buf[e, :]
                for k in range(HEADS):
                    wk = wv[k]
                    h_buf[e, pl.ds(16 * k, 16)] = h_buf[e, pl.ds(16 * k, 16)] * wk
                return c2

            lax.fori_loop(0, BL, mloop, 0)
            pltpu.sync_copy(h_buf, out_acc.at[dst_buf], add=True)
            return carry

        lax.fori_loop(0, chunks_per_tile, chunk, 0)
        plsc.subcore_barrier()

        for j in range(ROWS_PER_TILE // BL):
            r0 = base + j * BL
            pltpu.sync_copy(out_acc.at[pl.ds(r0, BL)],
                            numer_hbm.at[pl.ds(cid * NP + r0, BL)])
            pltpu.sync_copy(den_acc.at[pl.ds(r0, BL)],
                            denom_hbm.at[pl.ds(cid * NP + r0, BL)])

    return sc_agg


def _expand_att(a_src, a_dst):
    eye = jnp.eye(HEADS, dtype=jnp.float32)
    m_src = (a_src[:, :, None] * eye[:, None, :]).reshape(DIM, HEADS)
    m_dst = (a_dst[:, :, None] * eye[:, None, :]).reshape(DIM, HEADS)
    z = jnp.zeros((DIM, CHAN - HEADS), jnp.float32)
    return jnp.concatenate([m_src, z, m_dst, z], axis=1)


def kernel(x, edge_index, W1, a_src1, a_dst1, b1, W2, a_src2, a_dst2, b2,
           gamma1, beta1, gamma2, beta2):
    del gamma1, beta1, gamma2, beta2

    n = x.shape[0]
    e = edge_index.shape[1]
    total_edges = e + n
    chunks_per_tile = -(-total_edges // (NW * BL))
    ep = chunks_per_tile * NW * BL

    loops = jnp.arange(n, dtype=edge_index.dtype)
    src = jnp.concatenate([edge_index[0], loops])
    dst = jnp.concatenate([edge_index[1], loops])
    src = jnp.concatenate(
        [src, jnp.zeros((ep - total_edges,), edge_index.dtype)])
    dst = jnp.concatenate(
        [dst, jnp.full((ep - total_edges,), n, edge_index.dtype)])

    xp = jnp.zeros((NP, DIM), jnp.float32).at[:n].set(x)
    AM1 = _expand_att(a_src1, a_dst1)
    AM2 = _expand_att(a_src2, a_dst2)

    sc_agg = _make_sc_aggregate(chunks_per_tile)

    h1, asrc1, adst1 = _tc1(xp, W1, AM1)
    numer1, denom1 = sc_agg(h1, asrc1, adst1, src, dst)
    h2, asrc2, adst2 = _tc2(numer1, denom1, b1.reshape(1, DIM), W2, AM2)
    numer2, denom2 = sc_agg(h2, asrc2, adst2, src, dst)
    out = _tc3(numer2, denom2, b2.reshape(1, CHAN))
    return out[:n]


# speedup vs baseline: 126.4828x; 1.6538x over previous
import functools

import jax
import jax.numpy as jnp
from jax import lax
from jax.experimental import pallas as pl
from jax.experimental.pallas import tpu as pltpu
from jax.experimental.pallas import tpu_sc as plsc

N_NODES = 10000
DIM = 128
HEADS = 8
CHAN = 16
NP = 10240
BL = 64
NC, NS = 2, 16
NW = NC * NS
BN = 1024
ROWS_PER_TILE = NP // NS


def _hi_dot(a, b):
    return jnp.dot(a, b, preferred_element_type=jnp.float32,
                   precision=lax.Precision.HIGHEST)


def _tc1_body(x_ref, w_ref, am_ref, h_ref, asrc_ref, adst_ref):
    h = _hi_dot(x_ref[...], w_ref[...])
    h_ref[...] = h
    ab = _hi_dot(h, am_ref[...])
    asrc_ref[...] = ab[:, :CHAN]
    adst_ref[...] = ab[:, CHAN:]


def _tc1(xp, W, AM):
    return pl.pallas_call(
        _tc1_body,
        grid=(NP // BN,),
        in_specs=[
            pl.BlockSpec((BN, DIM), lambda i: (i, 0)),
            pl.BlockSpec((DIM, DIM), lambda i: (0, 0)),
            pl.BlockSpec((DIM, 2 * CHAN), lambda i: (0, 0)),
        ],
        out_specs=[
            pl.BlockSpec((BN, DIM), lambda i: (i, 0)),
            pl.BlockSpec((BN, CHAN), lambda i: (i, 0)),
            pl.BlockSpec((BN, CHAN), lambda i: (i, 0)),
        ],
        out_shape=[
            jax.ShapeDtypeStruct((NP, DIM), jnp.float32),
            jax.ShapeDtypeStruct((NP, CHAN), jnp.float32),
            jax.ShapeDtypeStruct((NP, CHAN), jnp.float32),
        ],
    )(xp, W, AM)


def _tc2_body(n0_ref, n1_ref, d0_ref, d1_ref, b_ref, w_ref, am_ref,
              h_ref, asrc_ref, adst_ref):
    num = n0_ref[...] + n1_ref[...]
    den = d0_ref[...] + d1_ref[...]
    cols = []
    for k in range(HEADS):
        cols.append(num[:, CHAN * k:CHAN * (k + 1)] /
                    (den[:, k:k + 1] + 1e-16))
    x1 = jnp.concatenate(cols, axis=1)
    x1 = jnp.maximum(x1 + b_ref[...], 0.0)
    h = _hi_dot(x1, w_ref[...])
    h_ref[...] = h
    ab = _hi_dot(h, am_ref[...])
    asrc_ref[...] = ab[:, :CHAN]
    adst_ref[...] = ab[:, CHAN:]


def _tc2(numer, denom, b1, W2, AM2):
    nblk = NP // BN
    return pl.pallas_call(
        _tc2_body,
        grid=(nblk,),
        in_specs=[
            pl.BlockSpec((BN, DIM), lambda i: (i, 0)),
            pl.BlockSpec((BN, DIM), lambda i, n=nblk: (n + i, 0)),
            pl.BlockSpec((BN, CHAN), lambda i: (i, 0)),
            pl.BlockSpec((BN, CHAN), lambda i, n=nblk: (n + i, 0)),
            pl.BlockSpec((1, DIM), lambda i: (0, 0)),
            pl.BlockSpec((DIM, DIM), lambda i: (0, 0)),
            pl.BlockSpec((DIM, 2 * CHAN), lambda i: (0, 0)),
        ],
        out_specs=[
            pl.BlockSpec((BN, DIM), lambda i: (i, 0)),
            pl.BlockSpec((BN, CHAN), lambda i: (i, 0)),
            pl.BlockSpec((BN, CHAN), lambda i: (i, 0)),
        ],
        out_shape=[
            jax.ShapeDtypeStruct((NP, DIM), jnp.float32),
            jax.ShapeDtypeStruct((NP, CHAN), jnp.float32),
            jax.ShapeDtypeStruct((NP, CHAN), jnp.float32),
        ],
    )(numer, numer, denom, denom, b1, W2, AM2)


def _tc3_body(n0_ref, n1_ref, d0_ref, d1_ref, b_ref, out_ref):
    num = n0_ref[...] + n1_ref[...]
    den = d0_ref[...] + d1_ref[...]
    acc = num[:, :CHAN] / (den[:, 0:1] + 1e-16)
    for k in range(1, HEADS):
        acc = acc + num[:, CHAN * k:CHAN * (k + 1)] / (den[:, k:k + 1] + 1e-16)
    out_ref[...] = acc * (1.0 / HEADS) + b_ref[...]


def _tc3(numer, denom, b2):
    nblk = NP // BN
    return pl.pallas_call(
        _tc3_body,
        grid=(nblk,),
        in_specs=[
            pl.BlockSpec((BN, DIM), lambda i: (i, 0)),
            pl.BlockSpec((BN, DIM), lambda i, n=nblk: (n + i, 0)),
            pl.BlockSpec((BN, CHAN), lambda i: (i, 0)),
            pl.BlockSpec((BN, CHAN), lambda i, n=nblk: (n + i, 0)),
            pl.BlockSpec((1, CHAN), lambda i: (0, 0)),
        ],
        out_specs=pl.BlockSpec((BN, CHAN), lambda i: (i, 0)),
        out_shape=jax.ShapeDtypeStruct((NP, CHAN), jnp.float32),
    )(numer, numer, denom, denom, b2)


NSLOT = 3


def _make_sc_aggregate(chunks_per_tile):
    assert chunks_per_tile % NSLOT == 0
    mesh = plsc.VectorSubcoreMesh(core_axis_name="c", subcore_axis_name="s",
                                  num_cores=NC, num_subcores=NS)

    scratch = [
        pltpu.VMEM_SHARED((NP, DIM), jnp.float32),
        pltpu.VMEM_SHARED((NP, CHAN), jnp.float32),
    ]
    scratch += [pltpu.VMEM((2, BL), jnp.int32)] * NSLOT
    scratch += [pltpu.VMEM((BL, CHAN), jnp.float32)] * NSLOT
    scratch += [pltpu.VMEM((BL, CHAN), jnp.float32)] * NSLOT
    scratch += [pltpu.VMEM((BL, CHAN), jnp.float32)] * NSLOT
    scratch += [pltpu.VMEM((BL, DIM), jnp.float32)] * NSLOT
    scratch += [pltpu.SemaphoreType.DMA] * (3 * NSLOT)

    @functools.partial(
        pl.kernel,
        out_type=(
            jax.ShapeDtypeStruct((NC * NP, DIM), jnp.float32),
            jax.ShapeDtypeStruct((NC * NP, CHAN), jnp.float32),
        ),
        mesh=mesh,
        compiler_params=pltpu.CompilerParams(use_tc_tiling_on_sc=False),
        scratch_types=tuple(scratch),
    )
    def sc_agg(h_hbm, asrc_hbm, adst_hbm, edges_hbm,
               numer_hbm, denom_hbm,
               out_acc, den_acc,
               idx0, idx1, idx2, as0, as1, as2, ad0, ad1, ad2,
               w0, w1, w2, hb0, hb1, hb2,
               si0, si1, si2, sg0, sg1, sg2, ss0, ss1, ss2):
        idx = (idx0, idx1, idx2)
        asr = (as0, as1, as2)
        adr = (ad0, ad1, ad2)
        wbf = (w0, w1, w2)
        hbf = (hb0, hb1, hb2)
        sem_i = (si0, si1, si2)
        sem_g = (sg0, sg1, sg2)
        sem_s = (ss0, ss1, ss2)

        cid = lax.axis_index("c")
        sid = lax.axis_index("s")
        wid = cid * NS + sid
        base = sid * ROWS_PER_TILE

        def issue_idx(g, s):
            pltpu.async_copy(edges_hbm.at[g], idx[s], sem_i[s])

        def wait_idx(s):
            pltpu.make_async_copy(edges_hbm.at[0], idx[s], sem_i[s]).wait()

        def issue_gathers(s):
            pltpu.async_copy(h_hbm.at[idx[s].at[0]], hbf[s], sem_g[s])
            pltpu.async_copy(asrc_hbm.at[idx[s].at[0]], asr[s], sem_g[s])
            pltpu.async_copy(adst_hbm.at[idx[s].at[1]], adr[s], sem_g[s])

        def wait_gathers(s):
            pltpu.make_async_copy(h_hbm.at[idx[s].at[0]], hbf[s], sem_g[s]).wait()
            pltpu.make_async_copy(asrc_hbm.at[idx[s].at[0]], asr[s], sem_g[s]).wait()
            pltpu.make_async_copy(adst_hbm.at[idx[s].at[1]], adr[s], sem_g[s]).wait()

        def issue_scatters(s):
            pltpu.async_copy(wbf[s], den_acc.at[idx[s].at[1]], sem_s[s], add=True)
            pltpu.async_copy(hbf[s], out_acc.at[idx[s].at[1]], sem_s[s], add=True)

        def wait_scatters(s):
            pltpu.make_async_copy(wbf[s], den_acc.at[idx[s].at[1]], sem_s[s]).wait()
            pltpu.make_async_copy(hbf[s], out_acc.at[idx[s].at[1]], sem_s[s]).wait()

        zero16 = jnp.zeros((16,), jnp.float32)

        def zrow(i, carry):
            for k in range(DIM // 16):
                hb0[i, pl.ds(16 * k, 16)] = zero16
            w0[i, :] = zero16
            return carry

        lax.fori_loop(0, BL, zrow, 0)
        for j in range(ROWS_PER_TILE // BL):
            pltpu.sync_copy(hb0, out_acc.at[pl.ds(base + j * BL, BL)])
            pltpu.sync_copy(w0, den_acc.at[pl.ds(base + j * BL, BL)])
        plsc.subcore_barrier()

        def compute(s):
            @plsc.parallel_loop(0, BL, unroll=2)
            def eloop(e):
                av = asr[s][e, :] + adr[s][e, :]
                av = jnp.where(av > 0.0, av, 0.2 * av)
                w = jnp.exp(av)
                wbf[s][e, :] = w
                for k in range(HEADS):
                    hk = hbf[s][e, pl.ds(16 * k, 16)]
                    hbf[s][e, pl.ds(16 * k, 16)] = hk * w[k]

        g0 = wid * chunks_per_tile

        issue_idx(g0, 0)
        issue_idx(g0 + 1, 1)
        wait_idx(0)
        issue_gathers(0)

        def body(t, carry):
            c0 = t * NSLOT
            for k in range(NSLOT):
                s = k
                s_next = (k + 1) % NSLOT
                s_prev = (k + 2) % NSLOT
                c = c0 + k
                wait_idx(s_next)
                issue_gathers(s_next)
                if k == 0:
                    @pl.when(t >= 1)
                    def _():
                        wait_scatters(s_prev)
                else:
                    wait_scatters(s_prev)
                issue_idx(g0 + c + 2, s_prev)
                wait_gathers(s)
                compute(s)
                issue_scatters(s)
            return carry

        lax.fori_loop(0, chunks_per_tile // NSLOT, body, 0)

        wait_scatters((chunks_per_tile - 1) % NSLOT)
        wait_gathers(chunks_per_tile % NSLOT)
        wait_idx((chunks_per_tile + 1) % NSLOT)
        plsc.subcore_barrier()

        for j in range(ROWS_PER_TILE // BL):
            r0 = base + j * BL
            pltpu.sync_copy(out_acc.at[pl.ds(r0, BL)],
                            numer_hbm.at[pl.ds(cid * NP + r0, BL)])
            pltpu.sync_copy(den_acc.at[pl.ds(r0, BL)],
                            denom_hbm.at[pl.ds(cid * NP + r0, BL)])

    return sc_agg


def _expand_att(a_src, a_dst):
    eye = jnp.eye(HEADS, dtype=jnp.float32)
    m_src = (a_src[:, :, None] * eye[:, None, :]).reshape(DIM, HEADS)
    m_dst = (a_dst[:, :, None] * eye[:, None, :]).reshape(DIM, HEADS)
    z = jnp.zeros((DIM, CHAN - HEADS), jnp.float32)
    return jnp.concatenate([m_src, z, m_dst, z], axis=1)


def kernel(x, edge_index, W1, a_src1, a_dst1, b1, W2, a_src2, a_dst2, b2,
           gamma1, beta1, gamma2, beta2):
    del gamma1, beta1, gamma2, beta2

    n = x.shape[0]
    e = edge_index.shape[1]
    total_edges = e + n
    chunks_per_tile = -(-total_edges // (NW * BL))
    chunks_per_tile = -(-chunks_per_tile // NSLOT) * NSLOT
    nch = chunks_per_tile * NW + 2
    ep = nch * BL

    loops = jnp.arange(n, dtype=edge_index.dtype)
    src = jnp.concatenate(
        [edge_index[0], loops,
         jnp.zeros((ep - total_edges,), edge_index.dtype)])
    dst = jnp.concatenate(
        [edge_index[1], loops,
         jnp.full((ep - total_edges,), n, edge_index.dtype)])
    edges3 = jnp.stack([src.reshape(nch, BL), dst.reshape(nch, BL)], axis=1)

    xp = jnp.zeros((NP, DIM), jnp.float32).at[:n].set(x)
    AM1 = _expand_att(a_src1, a_dst1)
    AM2 = _expand_att(a_src2, a_dst2)

    sc_agg = _make_sc_aggregate(chunks_per_tile)

    h1, asrc1, adst1 = _tc1(xp, W1, AM1)
    numer1, denom1 = sc_agg(h1, asrc1, adst1, edges3)
    h2, asrc2, adst2 = _tc2(numer1, denom1, b1.reshape(1, DIM), W2, AM2)
    numer2, denom2 = sc_agg(h2, asrc2, adst2, edges3)
    out = _tc3(numer2, denom2, b2.reshape(1, CHAN))
    return out[:n]
